# Initial kernel scaffold; baseline (speedup 1.0000x reference)
import jax
import jax.numpy as jnp
from jax.experimental import pallas as pl


def kernel(x, edge_index, edge_type, has_embedding, node_collections, jina_W, jina_b, type_emb, comb1, basis1, root1, bias1, comb2, basis2, root2, bias2):
    raise NotImplementedError("write your pallas kernel here")


# speedup vs baseline: 6.0098x; 6.0098x over previous
import functools

import jax
import jax.numpy as jnp
from jax import lax
from jax.experimental import pallas as pl
from jax.experimental.pallas import tpu as pltpu
from jax.experimental.pallas import tpu_sc as plsc

N = 10000
E = 160000
JINA = 768
H = 256
EMB = 128
R = 8
NB = 8
NTYPES = 62

NP = 10240
EP = 163840
RNP = R * NP
HALF = NP // 2
BN = 1024

CRNG = RNP // 16
CH_A = 2048
CH_B = 1024
EPT_B = EP // 32
CH_C = 1024
GB = 128
CAP = EP // 16 + GB
NQ = 2
QN = HALF // NQ


def _mesh():
    return plsc.VectorSubcoreMesh(core_axis_name="c", subcore_axis_name="s")


def _sc_count_w(rdst):
    @functools.partial(
        pl.kernel,
        out_type=jax.ShapeDtypeStruct((EP,), jnp.float32),
        mesh=_mesh(),
        compiler_params=pltpu.CompilerParams(needs_layout_passes=False),
        scratch_types=[
            pltpu.VMEM((CRNG,), jnp.float32),
            pltpu.VMEM((RNP,), jnp.float32),
            pltpu.VMEM((CH_A,), jnp.int32),
            pltpu.VMEM((CH_B,), jnp.float32),
            pltpu.VMEM_SHARED((RNP,), jnp.float32),
        ],
    )
    def k(rdst_hbm, w_hbm, cloc, cfull, ibuf, wbuf, csh):
        c = lax.axis_index("c")
        s = lax.axis_index("s")
        base = s * CRNG
        zero16 = jnp.zeros((16,), jnp.float32)
        ones16 = jnp.ones((16,), jnp.float32)

        def zbody(i, _):
            cloc[pl.ds(i * 16, 16)] = zero16
            return _
        lax.fori_loop(0, CRNG // 16, zbody, None)

        def chunk_a(ch, _):
            pltpu.sync_copy(rdst_hbm.at[pl.ds(ch * CH_A, CH_A)], ibuf)

            def vec_a(k2, _2):
                v = ibuf[pl.ds(k2 * 16, 16)]
                loc = v - base
                m = (loc >= 0) & (loc < CRNG)
                locc = jnp.clip(loc, 0, CRNG - 1)
                plsc.addupdate_scatter(cloc, [locc], ones16, mask=m)
                return _2
            lax.fori_loop(0, CH_A // 16, vec_a, None)
            return _
        lax.fori_loop(0, EP // CH_A, chunk_a, None)

        pltpu.sync_copy(cloc, csh.at[pl.ds(base, CRNG)])
        plsc.subcore_barrier()
        pltpu.sync_copy(csh, cfull)

        tid = c * 16 + s
        e0 = tid * EPT_B

        def chunk_b(ch, _):
            pltpu.sync_copy(rdst_hbm.at[pl.ds(e0 + ch * CH_B, CH_B)],
                            ibuf.at[pl.ds(0, CH_B)])

            def vec_b(k2, _2):
                v = ibuf[pl.ds(k2 * 16, 16)]
                cv = plsc.load_gather(cfull, [v])
                wbuf[pl.ds(k2 * 16, 16)] = 1.0 / jnp.maximum(cv, 1.0)
                return _2
            lax.fori_loop(0, CH_B // 16, vec_b, None)
            pltpu.sync_copy(wbuf, w_hbm.at[pl.ds(e0 + ch * CH_B, CH_B)])
            return _
        lax.fori_loop(0, EPT_B // CH_B, chunk_b, None)

    return k(rdst)


def _sc_edge(hr, gsrc2, dst2, w2, HD):
    @functools.partial(
        pl.kernel,
        out_type=jax.ShapeDtypeStruct((HD, NP, 128), jnp.float32),
        mesh=_mesh(),
        compiler_params=pltpu.CompilerParams(needs_layout_passes=False),
        scratch_types=(
            [
                pltpu.VMEM((8, 128), jnp.int32),
                pltpu.VMEM((8, 128), jnp.int32),
                pltpu.VMEM((8, 128), jnp.float32),
                pltpu.VMEM((CAP,), jnp.int32),
                pltpu.VMEM((CAP,), jnp.int32),
                pltpu.VMEM((CAP,), jnp.float32),
                pltpu.VMEM((GB,), jnp.int32),
                pltpu.VMEM((GB,), jnp.int32),
                pltpu.VMEM((32, 128), jnp.float32),
            ]
            + [pltpu.VMEM((GB, 128), jnp.float32) for _ in range(HD)]
            + [pltpu.VMEM_SHARED((QN, 128), jnp.float32) for _ in range(HD)]
        ),
    )
    def k(hr_hbm, gsrc_hbm, dst_hbm, w_hbm, out_hbm,
          sg, sd, sw, cg, cd, cw, bgid, bsid, zb, *bufs):
        bounces = bufs[:HD]
        accs = bufs[HD:]
        c = lax.axis_index("c")
        s = lax.axis_index("s")
        zero16 = jnp.zeros((16,), jnp.float32)
        izero16 = jnp.zeros((16,), jnp.int32)

        def zzb(i, _):
            zb[i // 8, pl.ds((i % 8) * 16, 16)] = zero16
            return _
        lax.fori_loop(0, 32 * 8, zzb, None)

        for q in range(NQ):
            nbase = c * HALF + q * QN

            def zacc(j, _):
                for h in range(HD):
                    pltpu.sync_copy(zb, accs[h].at[pl.ds(s * (QN // 16) + j * 32, 32)])
                return _
            lax.fori_loop(0, (QN // 16) // 32, zacc, None)
            plsc.subcore_barrier()

            r0 = s * (EP // 128 // 16)

            def chunk(ch, off):
                pltpu.sync_copy(gsrc_hbm.at[pl.ds(r0 + ch * 8, 8)], sg)
                pltpu.sync_copy(dst_hbm.at[pl.ds(r0 + ch * 8, 8)], sd)
                pltpu.sync_copy(w_hbm.at[pl.ds(r0 + ch * 8, 8)], sw)

                def vec(k2, off2):
                    row = k2 // 8
                    lane = (k2 % 8) * 16
                    dv = sd[row, pl.ds(lane, 16)]
                    gv = sg[row, pl.ds(lane, 16)]
                    wv = sw[row, pl.ds(lane, 16)]
                    dl = dv - nbase
                    m = (dl >= 0) & (dl < QN)
                    plsc.store_compressed(cg.at[pl.ds(off2, 16)], gv, mask=m)
                    plsc.store_compressed(cd.at[pl.ds(off2, 16)],
                                          jnp.clip(dl, 0, QN - 1), mask=m)
                    plsc.store_compressed(cw.at[pl.ds(off2, 16)], wv, mask=m)
                    return off2 + jnp.sum(m.astype(jnp.int32))
                return lax.fori_loop(0, 64, vec, off)
            off = lax.fori_loop(0, (EP // 16) // 1024, chunk, jnp.int32(0))

            def ztail(k2, _):
                cg[pl.ds(off + k2 * 16, 16)] = izero16
                cd[pl.ds(off + k2 * 16, 16)] = izero16
                cw[pl.ds(off + k2 * 16, 16)] = zero16
                return _
            lax.fori_loop(0, GB // 16, ztail, None)

            nblk = (off + GB - 1) // GB

            def blk(b, _):
                def stage_idx(k2, _2):
                    bgid[pl.ds(k2 * 16, 16)] = cg[pl.ds(b * GB + k2 * 16, 16)]
                    bsid[pl.ds(k2 * 16, 16)] = cd[pl.ds(b * GB + k2 * 16, 16)]
                    return _2
                lax.fori_loop(0, GB // 16, stage_idx, None)
                for h in range(HD):
                    pltpu.sync_copy(hr_hbm.at[h].at[bgid], bounces[h])

                def scale(j, _2):
                    wj = plsc.load_gather(
                        cw, [jnp.full((16,), b * GB + j, jnp.int32)])
                    for h in range(HD):
                        for p in range(8):
                            bounces[h][j, pl.ds(p * 16, 16)] = (
                                bounces[h][j, pl.ds(p * 16, 16)] * wj)
                    return _2
                lax.fori_loop(0, GB, scale, None)
                for h in range(HD):
                    pltpu.sync_copy(bounces[h], accs[h].at[bsid], add=True)
                return _
            lax.fori_loop(0, nblk, blk, None)

            plsc.subcore_barrier()
            for h in range(HD):
                pltpu.sync_copy(
                    accs[h].at[pl.ds(s * (QN // 16), QN // 16)],
                    out_hbm.at[h].at[pl.ds(nbase + s * (QN // 16), QN // 16)])
            plsc.subcore_barrier()

    return k(hr, gsrc2, dst2, w2)


def _tc_h0(xp, he, nc, jina_W, jina_b, type_emb):
    def body(x_ref, he_ref, nc_ref, w_ref, b_ref, te_ref, o_ref):
        xb = x_ref[...]
        n = jnp.sqrt(jnp.sum(xb * xb, axis=1, keepdims=True))
        inv = 1.0 / jnp.maximum(n, 1e-12)
        h = jnp.dot(xb, w_ref[...], preferred_element_type=jnp.float32) * inv
        h = h + b_ref[...]
        oh = (nc_ref[...] == lax.broadcasted_iota(jnp.int32, (BN, NTYPES), 1))
        te = jnp.dot(oh.astype(jnp.float32), te_ref[...],
                     preferred_element_type=jnp.float32)
        o_ref[...] = jnp.where(he_ref[...] > 0.0, h, te)

    return pl.pallas_call(
        body,
        grid=(NP // BN,),
        in_specs=[
            pl.BlockSpec((BN, JINA), lambda i: (i, 0)),
            pl.BlockSpec((BN, 1), lambda i: (i, 0)),
            pl.BlockSpec((BN, 1), lambda i: (i, 0)),
            pl.BlockSpec((JINA, H), lambda i: (0, 0)),
            pl.BlockSpec((1, H), lambda i: (0, 0)),
            pl.BlockSpec((NTYPES, H), lambda i: (0, 0)),
        ],
        out_specs=pl.BlockSpec((BN, H), lambda i: (i, 0)),
        out_shape=jax.ShapeDtypeStruct((NP, H), jnp.float32),
    )(xp, he, nc, jina_W, jina_b, type_emb)


def _tc_rel(h, comb, basis, D):
    def body(h_ref, comb_ref, basis_ref, o_ref, w_scr):
        r = pl.program_id(0)
        i = pl.program_id(1)

        @pl.when(i == 0)
        def _():
            acc = comb_ref[r, 0] * basis_ref[0]
            for b in range(1, NB):
                acc = acc + comb_ref[r, b] * basis_ref[b]
            w_scr[...] = acc

        y = jnp.dot(h_ref[...], w_scr[...], preferred_element_type=jnp.float32)
        for hh in range(D // 128):
            o_ref[hh] = y[:, hh * 128:(hh + 1) * 128]

    return pl.pallas_call(
        body,
        grid=(R, NP // BN),
        in_specs=[
            pl.BlockSpec((BN, H), lambda r, i: (i, 0)),
            pl.BlockSpec((R, NB), lambda r, i: (0, 0),
                         memory_space=pltpu.SMEM),
            pl.BlockSpec((NB, H, D), lambda r, i: (0, 0, 0)),
        ],
        out_specs=pl.BlockSpec((D // 128, BN, 128),
                               lambda r, i: (0, r * (NP // BN) + i, 0)),
        out_shape=jax.ShapeDtypeStruct((D // 128, RNP, 128), jnp.float32),
        scratch_shapes=[pltpu.VMEM((H, D), jnp.float32)],
    )(h, comb, basis)


def _tc_post(h, root, bias, agg, act, D):
    def body(h_ref, r_ref, b_ref, a_ref, o_ref):
        y = jnp.dot(h_ref[...], r_ref[...], preferred_element_type=jnp.float32)
        a = jnp.concatenate([a_ref[hh] for hh in range(D // 128)], axis=1)
        y = y + b_ref[...] + a
        if act == "relu":
            y = jnp.maximum(y, 0.0)
        else:
            n = jnp.sqrt(jnp.sum(y * y, axis=1, keepdims=True))
            y = y / jnp.maximum(n, 1e-12)
        o_ref[...] = y

    return pl.pallas_call(
        body,
        grid=(NP // BN,),
        in_specs=[
            pl.BlockSpec((BN, H), lambda i: (i, 0)),
            pl.BlockSpec((H, D), lambda i: (0, 0)),
            pl.BlockSpec((1, D), lambda i: (0, 0)),
            pl.BlockSpec((D // 128, BN, 128), lambda i: (0, i, 0)),
        ],
        out_specs=pl.BlockSpec((BN, D), lambda i: (i, 0)),
        out_shape=jax.ShapeDtypeStruct((NP, D), jnp.float32),
    )(h, root, bias, agg)


def kernel(x, edge_index, edge_type, has_embedding, node_collections,
           jina_W, jina_b, type_emb, comb1, basis1, root1, bias1,
           comb2, basis2, root2, bias2):
    src = edge_index[0]
    dst = edge_index[1]
    t = edge_type
    padE = EP - E
    gsrc = jnp.concatenate([t * NP + src, jnp.zeros((padE,), jnp.int32)])
    gsrc = gsrc.reshape(EP // 128, 128)
    dstp = jnp.concatenate([dst, jnp.full((padE,), N, jnp.int32)])
    dstp = dstp.reshape(EP // 128, 128)
    rdst = jnp.concatenate([t * NP + dst, jnp.full((padE,), RNP - 1, jnp.int32)])
    padN = NP - N
    xp = jnp.pad(x, ((0, padN), (0, 0)))
    he = jnp.pad(has_embedding.astype(jnp.float32), (0, padN)).reshape(NP, 1)
    nc = jnp.pad(node_collections, (0, padN)).reshape(NP, 1)

    w = _sc_count_w(rdst).reshape(EP // 128, 128)
    h0 = _tc_h0(xp, he, nc, jina_W, jina_b.reshape(1, H), type_emb)
    agg1 = _sc_edge(_tc_rel(h0, comb1, basis1, H), gsrc, dstp, w, 2)
    h1 = _tc_post(h0, root1, bias1.reshape(1, H), agg1, "relu", H)
    agg2 = _sc_edge(_tc_rel(h1, comb2, basis2, EMB), gsrc, dstp, w, 1)
    h2 = _tc_post(h1, root2, bias2.reshape(1, EMB), agg2, "l2", EMB)
    return h2[:N]
